# Initial kernel scaffold; baseline (speedup 1.0000x reference)
import jax
import jax.numpy as jnp
from jax.experimental import pallas as pl


def kernel(boxes, scores):
    raise NotImplementedError("write your pallas kernel here")


# speedup vs baseline: 116.6231x; 116.6231x over previous
import jax
import jax.numpy as jnp
from jax.experimental import pallas as pl
from jax.experimental.pallas import tpu as pltpu

_N = 20000
_ROWS = 160
_NP = _ROWS * 128
_K = 2000
_DET = 100
_NEG = -1e9
_PAD = -3e9
_DEAD = -2e9


def _nms_body(s_ref, x1_ref, y1_ref, x2_ref, y2_ref, out_ref):
    s = s_ref[...]
    key = jnp.where(s > jnp.float32(0.05), s, jnp.float32(_NEG))
    kbv = jax.lax.bitcast_convert_type(key, jnp.int32)
    ridx = jax.lax.broadcasted_iota(jnp.int32, (_ROWS, 128), 0)
    lidx = jax.lax.broadcasted_iota(jnp.int32, (_ROWS, 128), 1)
    idx = ridx * 128 + lidx

    lo0 = jax.lax.bitcast_convert_type(jnp.float32(0.04), jnp.int32)
    hi0 = jax.lax.bitcast_convert_type(jnp.float32(1.5), jnp.int32)

    def bs_body(_, lh):
        lo, hi = lh
        mid = (lo + hi) // 2
        cnt = jnp.sum(jnp.where(kbv >= mid, 1, 0))
        pred = cnt >= _K
        return (jnp.where(pred, mid, lo), jnp.where(pred, hi, mid))

    lo, hi = jax.lax.fori_loop(0, 26, bs_body, (lo0, hi0))

    n_gt = jnp.sum(jnp.where(kbv > lo, 1, 0))
    ties_needed = _K - n_gt

    def ib_body(_, lh):
        lom, him = lh
        mid = (lom + him) // 2
        cnt = jnp.sum(jnp.where((kbv == lo) & (idx < mid), 1, 0))
        pred = cnt >= ties_needed
        return (jnp.where(pred, lom, mid), jnp.where(pred, mid, him))

    lom, him = jax.lax.fori_loop(0, 15, ib_body, (jnp.int32(0), jnp.int32(_NP)))

    member = (kbv > lo) | ((kbv == lo) & (idx < him))
    key = jnp.where(member, key, jnp.float32(_PAD))

    x1 = x1_ref[...]
    y1 = y1_ref[...]
    x2 = x2_ref[...]
    y2 = y2_ref[...]
    area = (x2 - x1) * (y2 - y1)

    rowi = jax.lax.broadcasted_iota(jnp.int32, (8, 128), 0)
    lanei = jax.lax.broadcasted_iota(jnp.int32, (8, 128), 1)

    def g_body(t, carry):
        k, acc = carry
        m = jnp.max(k)
        sel = jnp.min(jnp.where(k == m, idx, jnp.int32(2**30)))
        onehot = idx == sel
        bx1 = jnp.sum(jnp.where(onehot, x1, jnp.float32(0.0)))
        by1 = jnp.sum(jnp.where(onehot, y1, jnp.float32(0.0)))
        bx2 = jnp.sum(jnp.where(onehot, x2, jnp.float32(0.0)))
        by2 = jnp.sum(jnp.where(onehot, y2, jnp.float32(0.0)))
        barea = jnp.sum(jnp.where(onehot, area, jnp.float32(0.0)))
        w = jnp.maximum(jnp.minimum(bx2, x2) - jnp.maximum(bx1, x1), jnp.float32(0.0))
        h = jnp.maximum(jnp.minimum(by2, y2) - jnp.maximum(by1, y1), jnp.float32(0.0))
        inter = w * h
        iou = inter / (((barea + area) - inter) + jnp.float32(1e-9))
        supp = (iou > jnp.float32(0.5)) & (k > jnp.float32(-5e8))
        nk = jnp.where(supp, k - jnp.float32(1e9), k)
        nk = jnp.where(onehot, jnp.float32(_DEAD), nk)
        outs = jnp.where(m > jnp.float32(-5e8), m, jnp.float32(_NEG))
        val = jnp.where(rowi == 0, bx1,
              jnp.where(rowi == 1, by1,
              jnp.where(rowi == 2, bx2,
              jnp.where(rowi == 3, by2, outs))))
        acc = jnp.where(lanei == t, val, acc)
        return nk, acc

    acc0 = jnp.zeros((8, 128), jnp.float32)
    _, acc = jax.lax.fori_loop(0, _DET, g_body, (key, acc0))
    out_ref[...] = acc


def kernel(boxes, scores):
    s = jnp.pad(scores, (0, _NP - _N), constant_values=-1.0).reshape(_ROWS, 128)
    b = jnp.pad(boxes, ((0, _NP - _N), (0, 0)))
    x1 = b[:, 0].reshape(_ROWS, 128)
    y1 = b[:, 1].reshape(_ROWS, 128)
    x2 = b[:, 2].reshape(_ROWS, 128)
    y2 = b[:, 3].reshape(_ROWS, 128)
    out = pl.pallas_call(
        _nms_body,
        out_shape=jax.ShapeDtypeStruct((8, 128), jnp.float32),
    )(s, x1, y1, x2, y2)
    return jnp.transpose(out[0:5, 0:_DET])
